# baseline (speedup 1.0000x reference)
import jax
import jax.numpy as jnp
import numpy as np
from jax import lax
from jax.experimental import pallas as pl
from jax.experimental.pallas import tpu as pltpu
from jax.experimental.pallas import tpu_sc as plsc

N = 10000
E = 320000
H = 128

NTILES = 32
CHUNK = 128
NCH = 80
EPT = NCH * CHUNK
EPAD = NTILES * EPT
NP = 10240
RPT = NP // 16

BN = 2000
BM = 2048

_relu = jax.nn.relu


def _bf(a):
    return a.astype(jnp.bfloat16)


def _dot(a, b):
    return jax.lax.dot_general(
        _bf(a), _bf(b), (((a.ndim - 1,), (0,)), ((), ())),
        preferred_element_type=jnp.float32)


def _dot_hi(a, b):
    return jax.lax.dot_general(
        a, b, (((a.ndim - 1,), (0,)), ((), ())),
        preferred_element_type=jnp.float32,
        precision=jax.lax.Precision.HIGHEST)


def _encoder_body(x, w1, bb1, w2, bb2, out):
    t = bb1[...] * jnp.ones((x.shape[0], 1), jnp.float32)
    for k in range(4):
        t = t + (_bf(x[:, k:k + 1]) * _bf(w1[k:k + 1, :])).astype(jnp.float32)
    t = _relu(t)
    out[...] = _dot(t, w2[...]) + bb2[...]


def _encoder(x, We1, be1, We2, be2):
    return pl.pallas_call(
        _encoder_body,
        grid=(N // BN,),
        in_specs=[
            pl.BlockSpec((BN, 4), lambda i: (i, 0)),
            pl.BlockSpec((4, H), lambda i: (0, 0)),
            pl.BlockSpec((1, H), lambda i: (0, 0)),
            pl.BlockSpec((H, H), lambda i: (0, 0)),
            pl.BlockSpec((1, H), lambda i: (0, 0)),
        ],
        out_specs=pl.BlockSpec((BN, H), lambda i: (i, 0)),
        out_shape=jax.ShapeDtypeStruct((N, H), jnp.float32),
    )(x, We1, be1[None], We2, be2[None])


def _node_prep_body(h, w1d, w1s, bb1, a, b):
    hv = h[...]
    a[...] = _dot(hv, w1d[...]) + bb1[...]
    b[...] = _dot(hv, w1s[...])


def _node_prep(h, W1d, W1s, b1):
    return pl.pallas_call(
        _node_prep_body,
        grid=(N // BN,),
        in_specs=[
            pl.BlockSpec((BN, H), lambda i: (i, 0)),
            pl.BlockSpec((H, H), lambda i: (0, 0)),
            pl.BlockSpec((H, H), lambda i: (0, 0)),
            pl.BlockSpec((1, H), lambda i: (0, 0)),
        ],
        out_specs=[
            pl.BlockSpec((BN, H), lambda i: (i, 0)),
            pl.BlockSpec((BN, H), lambda i: (i, 0)),
        ],
        out_shape=[
            jax.ShapeDtypeStruct((N, H), jnp.float32),
            jax.ShapeDtypeStruct((N, H), jnp.float32),
        ],
    )(h, W1d, W1s, b1[None])


def _edge_mm_body(ag, bg, ea, w1e, w2, bb2, w3, bb3, u):
    g = ag[...] + bg[...]
    for k in range(3):
        g = g + (_bf(ea[:, k:k + 1]) * _bf(w1e[k:k + 1, :])).astype(jnp.float32)
    e = _dot(_relu(g), w2[...]) + bb2[...]
    t = _dot(e, w3[...]) + bb3[...]
    u[...] = _bf(_relu(t)).astype(jnp.float32)


def _edge_mm(Ag, Bg, eap, W1e, W2, b2, W3, b3):
    return pl.pallas_call(
        _edge_mm_body,
        grid=(EPAD // BM,),
        in_specs=[
            pl.BlockSpec((BM, H), lambda i: (i, 0)),
            pl.BlockSpec((BM, H), lambda i: (i, 0)),
            pl.BlockSpec((BM, 3), lambda i: (i, 0)),
            pl.BlockSpec((3, H), lambda i: (0, 0)),
            pl.BlockSpec((H, H), lambda i: (0, 0)),
            pl.BlockSpec((1, H), lambda i: (0, 0)),
            pl.BlockSpec((H, H), lambda i: (0, 0)),
            pl.BlockSpec((1, H), lambda i: (0, 0)),
        ],
        out_specs=pl.BlockSpec((BM, H), lambda i: (i, 0)),
        out_shape=jax.ShapeDtypeStruct((EPAD, H), jnp.float32),
    )(Ag, Bg, eap, W1e, W2, b2[None], W3, b3[None])


def _node_update_body(h, s, cnt, wua, wub, w4, bb4, bbu1, wu2, bbu2, g, b, out):
    hv = h[...]
    ssum = s[0] + s[1]
    c = cnt[0, :, 0:1] + cnt[1, :, 0:1]
    aggr = _dot_hi(ssum, _bf(w4[...]).astype(jnp.float32)) + c * bb4[...]
    z = (_dot(hv, wua[...])
         + _dot(aggr, wub[...])
         + bbu1[...])
    hn = _dot(_relu(z), wu2[...]) + bbu2[...]
    y = hv + hn
    m = jnp.mean(y, axis=-1, keepdims=True)
    yc = y - m
    v = jnp.mean(yc * yc, axis=-1, keepdims=True)
    out[...] = yc * lax.rsqrt(v + 1e-5) * g[...] + b[...]


def _node_update(h, S, cnt, Wua, Wub, W4, b4, bu1, Wu2, bu2, lg, lb):
    full = lambda i: (0, 0)
    return pl.pallas_call(
        _node_update_body,
        grid=(N // BN,),
        in_specs=[
            pl.BlockSpec((BN, H), lambda i: (i, 0)),
            pl.BlockSpec((2, BN, H), lambda i: (0, i, 0)),
            pl.BlockSpec((2, BN, H), lambda i: (0, i, 0)),
            pl.BlockSpec((H, H), full),
            pl.BlockSpec((H, H), full),
            pl.BlockSpec((H, H), full),
            pl.BlockSpec((1, H), full),
            pl.BlockSpec((1, H), full),
            pl.BlockSpec((H, H), full),
            pl.BlockSpec((1, H), full),
            pl.BlockSpec((1, H), full),
            pl.BlockSpec((1, H), full),
        ],
        out_specs=pl.BlockSpec((BN, H), lambda i: (i, 0)),
        out_shape=jax.ShapeDtypeStruct((N, H), jnp.float32),
    )(h, S, cnt, Wua, Wub, W4, b4[None], bu1[None], Wu2, bu2[None], lg[None],
      lb[None])


def _pool_body(h, ap0, ab0, ap1r, ga, gb1, gbm, gb2, gg, gbb, wk, bk, wv, bv,
               kout, vout):
    hv = h[...]
    t = jnp.tanh(_dot(hv, ap0[...]) + ab0[...])
    logits = jnp.sum((_bf(t) * _bf(ap1r[...])).astype(jnp.float32),
                     axis=-1, keepdims=True)
    m = jnp.max(logits)
    ew = jnp.exp(logits - m)
    s = jnp.sum(ew)
    w = ew / s
    hg = jnp.sum(hv * w, axis=0, keepdims=True)
    for i in range(3):
        u = _relu(_dot(hg, ga[i])
                  + gb1[i:i + 1, :])
        u = _dot(u, gbm[i]) + gb2[i:i + 1, :]
        y = hg + u
        mu = jnp.mean(y, axis=-1, keepdims=True)
        yc = y - mu
        var = jnp.mean(yc * yc, axis=-1, keepdims=True)
        hg = yc * lax.rsqrt(var + 1e-5) * gg[...] + gbb[...]
    kout[...] = _dot(hg, wk[...]) + bk[...]
    vout[...] = _dot(hg, wv[...]) + bv[...]


def _pool_global(h, attn_pool, gmlp, ln_global, kp, vp):
    ap0, ab0 = attn_pool[0]
    ap1 = attn_pool[1][0]
    ga = jnp.stack([mp[0][0] for mp in gmlp])
    gb1 = jnp.stack([mp[0][1] for mp in gmlp])
    gbm = jnp.stack([mp[1][0] for mp in gmlp])
    gb2 = jnp.stack([mp[1][1] for mp in gmlp])
    gg, gbb = ln_global
    full2 = lambda: None
    return pl.pallas_call(
        _pool_body,
        grid=(1,),
        in_specs=[
            pl.BlockSpec((N, H), lambda i: (0, 0)),
            pl.BlockSpec((H, H), lambda i: (0, 0)),
            pl.BlockSpec((1, H), lambda i: (0, 0)),
            pl.BlockSpec((1, H), lambda i: (0, 0)),
            pl.BlockSpec((3, H, H), lambda i: (0, 0, 0)),
            pl.BlockSpec((3, H), lambda i: (0, 0)),
            pl.BlockSpec((3, H, H), lambda i: (0, 0, 0)),
            pl.BlockSpec((3, H), lambda i: (0, 0)),
            pl.BlockSpec((1, H), lambda i: (0, 0)),
            pl.BlockSpec((1, H), lambda i: (0, 0)),
            pl.BlockSpec((H, H), lambda i: (0, 0)),
            pl.BlockSpec((1, H), lambda i: (0, 0)),
            pl.BlockSpec((H, H), lambda i: (0, 0)),
            pl.BlockSpec((1, H), lambda i: (0, 0)),
        ],
        out_specs=[
            pl.BlockSpec((1, H), lambda i: (0, 0)),
            pl.BlockSpec((1, H), lambda i: (0, 0)),
        ],
        out_shape=[
            jax.ShapeDtypeStruct((1, H), jnp.float32),
            jax.ShapeDtypeStruct((1, H), jnp.float32),
        ],
    )(h, ap0, ab0[None], ap1.T, ga, gb1, gbm, gb2, gg[None], gbb[None],
      kp[0], kp[1][None], vp[0], vp[1][None])


def _fuse_body(h, kv, vv, wq, bq, wo1, bo1, wo2, bo2, wh1, bh1, wh2, bh2,
               o0, o1, o2):
    hv = h[...]
    q = _dot(hv, wq[...]) + bq[...]
    s = jnp.sum(q * kv[...], axis=-1, keepdims=True) * (1.0 / np.sqrt(H))
    w = jax.nn.sigmoid(s)
    xo = hv + w * vv[...]
    y = _relu(_dot(xo, wo1[...]) + bo1[...])
    y = _dot(y, wo2[...]) + bo2[...]
    outs = [o0, o1, o2]
    for n in range(3):
        yh = _relu(_dot(y, wh1[n])
                   + bh1[n:n + 1, :])
        outs[n][...] = (jnp.sum((_bf(yh) * _bf(wh2[n:n + 1, :])).astype(
            jnp.float32), axis=-1, keepdims=True) + bh2[n:n + 1, :])


def _fuse_heads(h, Kv, Vv, qp, outp, heads):
    wh1 = jnp.stack([heads[n][0][0] for n in ('switch', 'voltage', 'current')])
    bh1 = jnp.stack([heads[n][0][1] for n in ('switch', 'voltage', 'current')])
    wh2 = jnp.stack([heads[n][1][0][:, 0] for n in ('switch', 'voltage', 'current')])
    bh2 = jnp.stack([heads[n][1][1] for n in ('switch', 'voltage', 'current')])
    full = lambda i: (0, 0)
    return pl.pallas_call(
        _fuse_body,
        grid=(N // BN,),
        in_specs=[
            pl.BlockSpec((BN, H), lambda i: (i, 0)),
            pl.BlockSpec((1, H), full),
            pl.BlockSpec((1, H), full),
            pl.BlockSpec((H, H), full),
            pl.BlockSpec((1, H), full),
            pl.BlockSpec((H, H), full),
            pl.BlockSpec((1, H), full),
            pl.BlockSpec((H, H), full),
            pl.BlockSpec((1, H), full),
            pl.BlockSpec((3, H, 64), lambda i: (0, 0, 0)),
            pl.BlockSpec((3, 64), full),
            pl.BlockSpec((3, 64), full),
            pl.BlockSpec((3, 1), full),
        ],
        out_specs=[
            pl.BlockSpec((BN, 1), lambda i: (i, 0)),
            pl.BlockSpec((BN, 1), lambda i: (i, 0)),
            pl.BlockSpec((BN, 1), lambda i: (i, 0)),
        ],
        out_shape=[
            jax.ShapeDtypeStruct((N, 1), jnp.float32),
            jax.ShapeDtypeStruct((N, 1), jnp.float32),
            jax.ShapeDtypeStruct((N, 1), jnp.float32),
        ],
    )(h, Kv, Vv, qp[0], qp[1][None], outp[0][0], outp[0][1][None],
      outp[1][0], outp[1][1][None], wh1, bh1, wh2, bh2)


def _sc_mesh():
    return plsc.VectorSubcoreMesh(core_axis_name="c", subcore_axis_name="s")


def _sc_gather_body(a_hbm, b_hbm, di_hbm, si_hbm, ag_hbm, bg_hbm,
                    idx_d, idx_s, buf_a, buf_b, sem_a, sem_b):
    wid = lax.axis_index("s") * 2 + lax.axis_index("c")
    base = wid * EPT

    def body(j, carry):
        off = base + j * CHUNK
        pltpu.sync_copy(di_hbm.at[pl.ds(off, CHUNK)], idx_d)
        pltpu.sync_copy(si_hbm.at[pl.ds(off, CHUNK)], idx_s)
        ca = pltpu.async_copy(a_hbm.at[idx_d], buf_a, sem_a)
        cb = pltpu.async_copy(b_hbm.at[idx_s], buf_b, sem_b)
        ca.wait()
        cb.wait()
        pltpu.sync_copy(buf_a, ag_hbm.at[pl.ds(off, CHUNK)])
        pltpu.sync_copy(buf_b, bg_hbm.at[pl.ds(off, CHUNK)])
        return carry

    lax.fori_loop(0, NCH, body, 0)


def _sc_gather(A, B, dstp, srcp):
    return pl.kernel(
        _sc_gather_body,
        out_type=(jax.ShapeDtypeStruct((EPAD, H), jnp.float32),
                  jax.ShapeDtypeStruct((EPAD, H), jnp.float32)),
        mesh=_sc_mesh(),
        scratch_types=[
            pltpu.VMEM((CHUNK,), jnp.int32),
            pltpu.VMEM((CHUNK,), jnp.int32),
            pltpu.VMEM((CHUNK, H), jnp.float32),
            pltpu.VMEM((CHUNK, H), jnp.float32),
            pltpu.SemaphoreType.DMA,
            pltpu.SemaphoreType.DMA,
        ],
    )(A, B, dstp, srcp)


def _sc_scatter_body(u_hbm, ds_hbm, z_hbm, s_hbm, idx_v, buf_u, acc):
    cid = lax.axis_index("c")
    sid = lax.axis_index("s")
    wid = sid * 2 + cid
    rbase = sid * RPT
    pltpu.sync_copy(z_hbm.at[pl.ds(rbase, RPT)], acc.at[pl.ds(rbase, RPT)])
    plsc.subcore_barrier()
    base = wid * EPT

    def body(j, carry):
        off = base + j * CHUNK
        pltpu.sync_copy(ds_hbm.at[pl.ds(off, CHUNK)], idx_v)
        pltpu.sync_copy(u_hbm.at[pl.ds(off, CHUNK)], buf_u)
        pltpu.sync_copy(buf_u, acc.at[idx_v], add=True)
        return carry

    lax.fori_loop(0, NCH, body, 0)
    plsc.subcore_barrier()
    pltpu.sync_copy(acc.at[pl.ds(rbase, RPT)], s_hbm.at[cid].at[pl.ds(rbase, RPT)])


def _sc_scatter(U, dsts, zf):
    return pl.kernel(
        _sc_scatter_body,
        out_type=jax.ShapeDtypeStruct((2, NP, H), jnp.float32),
        mesh=_sc_mesh(),
        scratch_types=[
            pltpu.VMEM((CHUNK,), jnp.int32),
            pltpu.VMEM((CHUNK, H), jnp.float32),
            pltpu.VMEM_SHARED((NP, H), jnp.float32),
        ],
    )(U, dsts, zf)


def _sc_cnt_body(ds_hbm, z_hbm, ones_hbm, cnt_hbm, idx_v, ones_v, acc):
    cid = lax.axis_index("c")
    sid = lax.axis_index("s")
    wid = sid * 2 + cid
    rbase = sid * RPT
    pltpu.sync_copy(z_hbm.at[pl.ds(rbase, RPT)], acc.at[pl.ds(rbase, RPT)])
    pltpu.sync_copy(ones_hbm, ones_v)
    plsc.subcore_barrier()
    base = wid * EPT

    def body(j, carry):
        off = base + j * CHUNK
        pltpu.sync_copy(ds_hbm.at[pl.ds(off, CHUNK)], idx_v)
        pltpu.sync_copy(ones_v, acc.at[idx_v], add=True)
        return carry

    lax.fori_loop(0, NCH, body, 0)
    plsc.subcore_barrier()
    pltpu.sync_copy(acc.at[pl.ds(rbase, RPT)], cnt_hbm.at[cid].at[pl.ds(rbase, RPT)])


def _sc_cnt(dsts, zc, ones):
    return pl.kernel(
        _sc_cnt_body,
        out_type=jax.ShapeDtypeStruct((2, NP, H), jnp.float32),
        mesh=_sc_mesh(),
        scratch_types=[
            pltpu.VMEM((CHUNK,), jnp.int32),
            pltpu.VMEM((CHUNK, H), jnp.float32),
            pltpu.VMEM_SHARED((NP, H), jnp.float32),
        ],
    )(dsts, zc, ones)


def kernel(x, edge_index, edge_attr, params):
    p = params
    src = edge_index[0]
    dst = edge_index[1]
    padz = jnp.zeros((EPAD - E,), jnp.int32)
    srcp = jnp.concatenate([src, padz])
    dstp = jnp.concatenate([dst, padz])
    dsts = jnp.concatenate([dst, jnp.full((EPAD - E,), N, jnp.int32)])
    eap = jnp.concatenate(
        [edge_attr, jnp.zeros((EPAD - E, 3), edge_attr.dtype)], axis=0)
    zf = jnp.zeros((NP, H), jnp.float32)
    ones128 = jnp.ones((CHUNK, H), jnp.float32)

    convs = p['convs']
    W1 = jnp.stack([c['edge'][0][0] for c in convs])
    b1 = jnp.stack([c['edge'][0][1] for c in convs])
    W2 = jnp.stack([c['edge'][1][0] for c in convs])
    b2 = jnp.stack([c['edge'][1][1] for c in convs])
    W3 = jnp.stack([c['msg'][0][0] for c in convs])
    b3 = jnp.stack([c['msg'][0][1] for c in convs])
    W4 = jnp.stack([c['msg'][1][0] for c in convs])
    b4 = jnp.stack([c['msg'][1][1] for c in convs])
    Wu = jnp.stack([c['upd'][0][0] for c in convs])
    bu1 = jnp.stack([c['upd'][0][1] for c in convs])
    Wu2 = jnp.stack([c['upd'][1][0] for c in convs])
    bu2 = jnp.stack([c['upd'][1][1] for c in convs])
    W1d = W1[:, :H]
    W1s = W1[:, H:2 * H]
    W1e = W1[:, 2 * H:]
    Wua = Wu[:, :H]
    Wub = Wu[:, H:]

    h = _encoder(x, p['enc'][0][0], p['enc'][0][1], p['enc'][1][0], p['enc'][1][1])

    cnt = _sc_cnt(dsts, zf, ones128)

    lg, lb = p['ln_local']
    for l in range(4):
        A, B = _node_prep(h, W1d[l], W1s[l], b1[l])
        Ag, Bg = _sc_gather(A, B, dstp, srcp)
        U = _edge_mm(Ag, Bg, eap, W1e[l], W2[l], b2[l], W3[l], b3[l])
        S = _sc_scatter(U, dsts, zf)
        h = _node_update(h, S, cnt, Wua[l], Wub[l], W4[l], b4[l], bu1[l],
                         Wu2[l], bu2[l], lg, lb)

    Kv, Vv = _pool_global(h, p['attn_pool'], p['gmlp'], p['ln_global'],
                          p['k'], p['v'])
    o0, o1, o2 = _fuse_heads(h, Kv, Vv, p['q'], p['outp'], p['heads'])
    return jnp.concatenate([o0, o1, o2], axis=-1)


# speedup vs baseline: 1.2824x; 1.2824x over previous
import jax
import jax.numpy as jnp
import numpy as np
from jax import lax
from jax.experimental import pallas as pl
from jax.experimental.pallas import tpu as pltpu
from jax.experimental.pallas import tpu_sc as plsc

N = 10000
E = 320000
H = 128

NTILES = 32
CHUNK = 128
NCH = 79
EPT = NCH * CHUNK
EPAD = NTILES * EPT
NP = 10240
RPT = NP // 16

BN = 2000
BM = 2048

_relu = jax.nn.relu


def _bf(a):
    return a.astype(jnp.bfloat16)


def _dot(a, b):
    return jax.lax.dot_general(
        _bf(a), _bf(b), (((a.ndim - 1,), (0,)), ((), ())),
        preferred_element_type=jnp.float32)


def _dot_hi(a, b):
    return jax.lax.dot_general(
        a, b, (((a.ndim - 1,), (0,)), ((), ())),
        preferred_element_type=jnp.float32,
        precision=jax.lax.Precision.HIGHEST)


def _encoder_body(x, w1, bb1, w2, bb2, out):
    t = bb1[...] * jnp.ones((x.shape[0], 1), jnp.float32)
    for k in range(4):
        t = t + (_bf(x[:, k:k + 1]) * _bf(w1[k:k + 1, :])).astype(jnp.float32)
    t = _relu(t)
    out[...] = _dot(t, w2[...]) + bb2[...]


def _encoder(x, We1, be1, We2, be2):
    return pl.pallas_call(
        _encoder_body,
        grid=(N // BN,),
        in_specs=[
            pl.BlockSpec((BN, 4), lambda i: (i, 0)),
            pl.BlockSpec((4, H), lambda i: (0, 0)),
            pl.BlockSpec((1, H), lambda i: (0, 0)),
            pl.BlockSpec((H, H), lambda i: (0, 0)),
            pl.BlockSpec((1, H), lambda i: (0, 0)),
        ],
        out_specs=pl.BlockSpec((BN, H), lambda i: (i, 0)),
        out_shape=jax.ShapeDtypeStruct((N, H), jnp.float32),
    )(x, We1, be1[None], We2, be2[None])


def _node_prep_body(h, w1d, w1s, bb1, a, b):
    hv = h[...]
    a[...] = _dot(hv, w1d[...]) + bb1[...]
    b[...] = _dot(hv, w1s[...])


def _node_prep(h, W1d, W1s, b1):
    return pl.pallas_call(
        _node_prep_body,
        grid=(N // BN,),
        in_specs=[
            pl.BlockSpec((BN, H), lambda i: (i, 0)),
            pl.BlockSpec((H, H), lambda i: (0, 0)),
            pl.BlockSpec((H, H), lambda i: (0, 0)),
            pl.BlockSpec((1, H), lambda i: (0, 0)),
        ],
        out_specs=[
            pl.BlockSpec((BN, H), lambda i: (i, 0)),
            pl.BlockSpec((BN, H), lambda i: (i, 0)),
        ],
        out_shape=[
            jax.ShapeDtypeStruct((N, H), jnp.float32),
            jax.ShapeDtypeStruct((N, H), jnp.float32),
        ],
    )(h, W1d, W1s, b1[None])


def _edge_mm_body(ag, bg, ea, w1e, w2, bb2, w3, bb3, u):
    g = ag[...] + bg[...]
    for k in range(3):
        g = g + (_bf(ea[:, k:k + 1]) * _bf(w1e[k:k + 1, :])).astype(jnp.float32)
    e = _dot(_relu(g), w2[...]) + bb2[...]
    t = _dot(e, w3[...]) + bb3[...]
    u[...] = _bf(_relu(t)).astype(jnp.float32)


def _edge_mm(Ag, Bg, eap, W1e, W2, b2, W3, b3):
    return pl.pallas_call(
        _edge_mm_body,
        grid=(EPAD // BM,),
        in_specs=[
            pl.BlockSpec((BM, H), lambda i: (i, 0)),
            pl.BlockSpec((BM, H), lambda i: (i, 0)),
            pl.BlockSpec((BM, 3), lambda i: (i, 0)),
            pl.BlockSpec((3, H), lambda i: (0, 0)),
            pl.BlockSpec((H, H), lambda i: (0, 0)),
            pl.BlockSpec((1, H), lambda i: (0, 0)),
            pl.BlockSpec((H, H), lambda i: (0, 0)),
            pl.BlockSpec((1, H), lambda i: (0, 0)),
        ],
        out_specs=pl.BlockSpec((BM, H), lambda i: (i, 0)),
        out_shape=jax.ShapeDtypeStruct((EPAD, H), jnp.float32),
    )(Ag, Bg, eap, W1e, W2, b2[None], W3, b3[None])


def _node_update_body(h, s, cnt, wua, wub, w4, bb4, bbu1, wu2, bbu2, g, b, out):
    hv = h[...]
    ssum = s[0] + s[1]
    c = cnt[0, :, 0:1] + cnt[1, :, 0:1]
    aggr = _dot_hi(ssum, _bf(w4[...]).astype(jnp.float32)) + c * bb4[...]
    z = (_dot(hv, wua[...])
         + _dot(aggr, wub[...])
         + bbu1[...])
    hn = _dot(_relu(z), wu2[...]) + bbu2[...]
    y = hv + hn
    m = jnp.mean(y, axis=-1, keepdims=True)
    yc = y - m
    v = jnp.mean(yc * yc, axis=-1, keepdims=True)
    out[...] = yc * lax.rsqrt(v + 1e-5) * g[...] + b[...]


def _node_update(h, S, cnt, Wua, Wub, W4, b4, bu1, Wu2, bu2, lg, lb):
    full = lambda i: (0, 0)
    return pl.pallas_call(
        _node_update_body,
        grid=(N // BN,),
        in_specs=[
            pl.BlockSpec((BN, H), lambda i: (i, 0)),
            pl.BlockSpec((2, BN, H), lambda i: (0, i, 0)),
            pl.BlockSpec((2, BN, H), lambda i: (0, i, 0)),
            pl.BlockSpec((H, H), full),
            pl.BlockSpec((H, H), full),
            pl.BlockSpec((H, H), full),
            pl.BlockSpec((1, H), full),
            pl.BlockSpec((1, H), full),
            pl.BlockSpec((H, H), full),
            pl.BlockSpec((1, H), full),
            pl.BlockSpec((1, H), full),
            pl.BlockSpec((1, H), full),
        ],
        out_specs=pl.BlockSpec((BN, H), lambda i: (i, 0)),
        out_shape=jax.ShapeDtypeStruct((N, H), jnp.float32),
    )(h, S, cnt, Wua, Wub, W4, b4[None], bu1[None], Wu2, bu2[None], lg[None],
      lb[None])


def _pool_body(h, ap0, ab0, ap1r, ga, gb1, gbm, gb2, gg, gbb, wk, bk, wv, bv,
               kout, vout):
    hv = h[...]
    t = jnp.tanh(_dot(hv, ap0[...]) + ab0[...])
    logits = jnp.sum((_bf(t) * _bf(ap1r[...])).astype(jnp.float32),
                     axis=-1, keepdims=True)
    m = jnp.max(logits)
    ew = jnp.exp(logits - m)
    s = jnp.sum(ew)
    w = ew / s
    hg = jnp.sum(hv * w, axis=0, keepdims=True)
    for i in range(3):
        u = _relu(_dot(hg, ga[i])
                  + gb1[i:i + 1, :])
        u = _dot(u, gbm[i]) + gb2[i:i + 1, :]
        y = hg + u
        mu = jnp.mean(y, axis=-1, keepdims=True)
        yc = y - mu
        var = jnp.mean(yc * yc, axis=-1, keepdims=True)
        hg = yc * lax.rsqrt(var + 1e-5) * gg[...] + gbb[...]
    kout[...] = _dot(hg, wk[...]) + bk[...]
    vout[...] = _dot(hg, wv[...]) + bv[...]


def _pool_global(h, attn_pool, gmlp, ln_global, kp, vp):
    ap0, ab0 = attn_pool[0]
    ap1 = attn_pool[1][0]
    ga = jnp.stack([mp[0][0] for mp in gmlp])
    gb1 = jnp.stack([mp[0][1] for mp in gmlp])
    gbm = jnp.stack([mp[1][0] for mp in gmlp])
    gb2 = jnp.stack([mp[1][1] for mp in gmlp])
    gg, gbb = ln_global
    full2 = lambda: None
    return pl.pallas_call(
        _pool_body,
        grid=(1,),
        in_specs=[
            pl.BlockSpec((N, H), lambda i: (0, 0)),
            pl.BlockSpec((H, H), lambda i: (0, 0)),
            pl.BlockSpec((1, H), lambda i: (0, 0)),
            pl.BlockSpec((1, H), lambda i: (0, 0)),
            pl.BlockSpec((3, H, H), lambda i: (0, 0, 0)),
            pl.BlockSpec((3, H), lambda i: (0, 0)),
            pl.BlockSpec((3, H, H), lambda i: (0, 0, 0)),
            pl.BlockSpec((3, H), lambda i: (0, 0)),
            pl.BlockSpec((1, H), lambda i: (0, 0)),
            pl.BlockSpec((1, H), lambda i: (0, 0)),
            pl.BlockSpec((H, H), lambda i: (0, 0)),
            pl.BlockSpec((1, H), lambda i: (0, 0)),
            pl.BlockSpec((H, H), lambda i: (0, 0)),
            pl.BlockSpec((1, H), lambda i: (0, 0)),
        ],
        out_specs=[
            pl.BlockSpec((1, H), lambda i: (0, 0)),
            pl.BlockSpec((1, H), lambda i: (0, 0)),
        ],
        out_shape=[
            jax.ShapeDtypeStruct((1, H), jnp.float32),
            jax.ShapeDtypeStruct((1, H), jnp.float32),
        ],
    )(h, ap0, ab0[None], ap1.T, ga, gb1, gbm, gb2, gg[None], gbb[None],
      kp[0], kp[1][None], vp[0], vp[1][None])


def _fuse_body(h, kv, vv, wq, bq, wo1, bo1, wo2, bo2, wh1, bh1, wh2, bh2,
               o0, o1, o2):
    hv = h[...]
    q = _dot(hv, wq[...]) + bq[...]
    s = jnp.sum(q * kv[...], axis=-1, keepdims=True) * (1.0 / np.sqrt(H))
    w = jax.nn.sigmoid(s)
    xo = hv + w * vv[...]
    y = _relu(_dot(xo, wo1[...]) + bo1[...])
    y = _dot(y, wo2[...]) + bo2[...]
    outs = [o0, o1, o2]
    for n in range(3):
        yh = _relu(_dot(y, wh1[n])
                   + bh1[n:n + 1, :])
        outs[n][...] = (jnp.sum((_bf(yh) * _bf(wh2[n:n + 1, :])).astype(
            jnp.float32), axis=-1, keepdims=True) + bh2[n:n + 1, :])


def _fuse_heads(h, Kv, Vv, qp, outp, heads):
    wh1 = jnp.stack([heads[n][0][0] for n in ('switch', 'voltage', 'current')])
    bh1 = jnp.stack([heads[n][0][1] for n in ('switch', 'voltage', 'current')])
    wh2 = jnp.stack([heads[n][1][0][:, 0] for n in ('switch', 'voltage', 'current')])
    bh2 = jnp.stack([heads[n][1][1] for n in ('switch', 'voltage', 'current')])
    full = lambda i: (0, 0)
    return pl.pallas_call(
        _fuse_body,
        grid=(N // BN,),
        in_specs=[
            pl.BlockSpec((BN, H), lambda i: (i, 0)),
            pl.BlockSpec((1, H), full),
            pl.BlockSpec((1, H), full),
            pl.BlockSpec((H, H), full),
            pl.BlockSpec((1, H), full),
            pl.BlockSpec((H, H), full),
            pl.BlockSpec((1, H), full),
            pl.BlockSpec((H, H), full),
            pl.BlockSpec((1, H), full),
            pl.BlockSpec((3, H, 64), lambda i: (0, 0, 0)),
            pl.BlockSpec((3, 64), full),
            pl.BlockSpec((3, 64), full),
            pl.BlockSpec((3, 1), full),
        ],
        out_specs=[
            pl.BlockSpec((BN, 1), lambda i: (i, 0)),
            pl.BlockSpec((BN, 1), lambda i: (i, 0)),
            pl.BlockSpec((BN, 1), lambda i: (i, 0)),
        ],
        out_shape=[
            jax.ShapeDtypeStruct((N, 1), jnp.float32),
            jax.ShapeDtypeStruct((N, 1), jnp.float32),
            jax.ShapeDtypeStruct((N, 1), jnp.float32),
        ],
    )(h, Kv, Vv, qp[0], qp[1][None], outp[0][0], outp[0][1][None],
      outp[1][0], outp[1][1][None], wh1, bh1, wh2, bh2)


def _sc_mesh():
    return plsc.VectorSubcoreMesh(core_axis_name="c", subcore_axis_name="s")


def _sc_gather_body(a_hbm, b_hbm, di_hbm, si_hbm, ag_hbm, bg_hbm,
                    idx_d, idx_s, buf_a, buf_b, sem_a, sem_b):
    wid = lax.axis_index("s") * 2 + lax.axis_index("c")
    base = wid * EPT

    def body(j, carry):
        off = base + j * CHUNK
        pltpu.sync_copy(di_hbm.at[pl.ds(off, CHUNK)], idx_d)
        pltpu.sync_copy(si_hbm.at[pl.ds(off, CHUNK)], idx_s)
        ca = pltpu.async_copy(a_hbm.at[idx_d], buf_a, sem_a)
        cb = pltpu.async_copy(b_hbm.at[idx_s], buf_b, sem_b)
        ca.wait()
        cb.wait()
        pltpu.sync_copy(buf_a, ag_hbm.at[pl.ds(off, CHUNK)])
        pltpu.sync_copy(buf_b, bg_hbm.at[pl.ds(off, CHUNK)])
        return carry

    lax.fori_loop(0, NCH, body, 0)


def _sc_gather(A, B, dstp, srcp):
    return pl.kernel(
        _sc_gather_body,
        out_type=(jax.ShapeDtypeStruct((EPAD, H), jnp.float32),
                  jax.ShapeDtypeStruct((EPAD, H), jnp.float32)),
        mesh=_sc_mesh(),
        scratch_types=[
            pltpu.VMEM((CHUNK,), jnp.int32),
            pltpu.VMEM((CHUNK,), jnp.int32),
            pltpu.VMEM((CHUNK, H), jnp.float32),
            pltpu.VMEM((CHUNK, H), jnp.float32),
            pltpu.SemaphoreType.DMA,
            pltpu.SemaphoreType.DMA,
        ],
    )(A, B, dstp, srcp)


def _sc_scatter_body(u_hbm, ds_hbm, z_hbm, s_hbm, idx_v, buf_u, acc):
    cid = lax.axis_index("c")
    sid = lax.axis_index("s")
    wid = sid * 2 + cid
    rbase = sid * RPT
    pltpu.sync_copy(z_hbm.at[pl.ds(rbase, RPT)], acc.at[pl.ds(rbase, RPT)])
    plsc.subcore_barrier()
    base = wid * EPT

    def body(j, carry):
        off = base + j * CHUNK
        pltpu.sync_copy(ds_hbm.at[pl.ds(off, CHUNK)], idx_v)
        pltpu.sync_copy(u_hbm.at[pl.ds(off, CHUNK)], buf_u)
        pltpu.sync_copy(buf_u, acc.at[idx_v], add=True)
        return carry

    lax.fori_loop(0, NCH, body, 0)
    plsc.subcore_barrier()
    pltpu.sync_copy(acc.at[pl.ds(rbase, RPT)], s_hbm.at[cid].at[pl.ds(rbase, RPT)])


def _sc_scatter(U, dsts, zf):
    return pl.kernel(
        _sc_scatter_body,
        out_type=jax.ShapeDtypeStruct((2, NP, H), jnp.float32),
        mesh=_sc_mesh(),
        scratch_types=[
            pltpu.VMEM((CHUNK,), jnp.int32),
            pltpu.VMEM((CHUNK, H), jnp.float32),
            pltpu.VMEM_SHARED((NP, H), jnp.float32),
        ],
    )(U, dsts, zf)


def _sc_cnt_body(ds_hbm, z_hbm, ones_hbm, cnt_hbm, idx_v, ones_v, acc):
    cid = lax.axis_index("c")
    sid = lax.axis_index("s")
    wid = sid * 2 + cid
    rbase = sid * RPT
    pltpu.sync_copy(z_hbm.at[pl.ds(rbase, RPT)], acc.at[pl.ds(rbase, RPT)])
    pltpu.sync_copy(ones_hbm, ones_v)
    plsc.subcore_barrier()
    base = wid * EPT

    def body(j, carry):
        off = base + j * CHUNK
        pltpu.sync_copy(ds_hbm.at[pl.ds(off, CHUNK)], idx_v)
        pltpu.sync_copy(ones_v, acc.at[idx_v], add=True)
        return carry

    lax.fori_loop(0, NCH, body, 0)
    plsc.subcore_barrier()
    pltpu.sync_copy(acc.at[pl.ds(rbase, RPT)], cnt_hbm.at[cid].at[pl.ds(rbase, RPT)])


def _sc_cnt(dsts, zc, ones):
    return pl.kernel(
        _sc_cnt_body,
        out_type=jax.ShapeDtypeStruct((2, NP, H), jnp.float32),
        mesh=_sc_mesh(),
        scratch_types=[
            pltpu.VMEM((CHUNK,), jnp.int32),
            pltpu.VMEM((CHUNK, H), jnp.float32),
            pltpu.VMEM_SHARED((NP, H), jnp.float32),
        ],
    )(dsts, zc, ones)


def kernel(x, edge_index, edge_attr, params):
    p = params
    src = edge_index[0]
    dst = edge_index[1]
    padz = jnp.zeros((EPAD - E,), jnp.int32)
    srcp = jnp.concatenate([src, padz])
    dstp = jnp.concatenate([dst, padz])
    dsts = jnp.concatenate([dst, jnp.full((EPAD - E,), N, jnp.int32)])
    eap = jnp.concatenate(
        [edge_attr, jnp.zeros((EPAD - E, 3), edge_attr.dtype)], axis=0)
    zf = jnp.zeros((NP, H), jnp.float32)
    ones128 = jnp.ones((CHUNK, H), jnp.float32)

    convs = p['convs']
    W1 = jnp.stack([c['edge'][0][0] for c in convs])
    b1 = jnp.stack([c['edge'][0][1] for c in convs])
    W2 = jnp.stack([c['edge'][1][0] for c in convs])
    b2 = jnp.stack([c['edge'][1][1] for c in convs])
    W3 = jnp.stack([c['msg'][0][0] for c in convs])
    b3 = jnp.stack([c['msg'][0][1] for c in convs])
    W4 = jnp.stack([c['msg'][1][0] for c in convs])
    b4 = jnp.stack([c['msg'][1][1] for c in convs])
    Wu = jnp.stack([c['upd'][0][0] for c in convs])
    bu1 = jnp.stack([c['upd'][0][1] for c in convs])
    Wu2 = jnp.stack([c['upd'][1][0] for c in convs])
    bu2 = jnp.stack([c['upd'][1][1] for c in convs])
    W1d = W1[:, :H]
    W1s = W1[:, H:2 * H]
    W1e = W1[:, 2 * H:]
    Wua = Wu[:, :H]
    Wub = Wu[:, H:]

    h = _encoder(x, p['enc'][0][0], p['enc'][0][1], p['enc'][1][0], p['enc'][1][1])

    cnt = _sc_cnt(dsts, zf, ones128)

    lg, lb = p['ln_local']
    for l in range(4):
        A, B = _node_prep(h, W1d[l], W1s[l], b1[l])
        Ag, Bg = _sc_gather(A, B, dstp, srcp)
        U = _edge_mm(Ag, Bg, eap, W1e[l], W2[l], b2[l], W3[l], b3[l])
        S = _sc_scatter(U, dsts, zf)
        h = _node_update(h, S, cnt, Wua[l], Wub[l], W4[l], b4[l], bu1[l],
                         Wu2[l], bu2[l], lg, lb)

    Kv, Vv = _pool_global(h, p['attn_pool'], p['gmlp'], p['ln_global'],
                          p['k'], p['v'])
    o0, o1, o2 = _fuse_heads(h, Kv, Vv, p['q'], p['outp'], p['heads'])
    return jnp.concatenate([o0, o1, o2], axis=-1)


# speedup vs baseline: 1.6316x; 1.2723x over previous
import jax
import jax.numpy as jnp
import numpy as np
from jax import lax
from jax.experimental import pallas as pl
from jax.experimental.pallas import tpu as pltpu
from jax.experimental.pallas import tpu_sc as plsc

N = 10000
E = 320000
H = 128

NTILES = 32
CHUNK = 128
NCH = 79
EPT = NCH * CHUNK
EPAD = NTILES * EPT
NP = 10240
RPT = NP // 16

BN = 2000
BM = 2048

_relu = jax.nn.relu


def _bf(a):
    return a.astype(jnp.bfloat16)


def _dot(a, b):
    return jax.lax.dot_general(
        _bf(a), _bf(b), (((a.ndim - 1,), (0,)), ((), ())),
        preferred_element_type=jnp.float32)


def _dot_hi(a, b):
    return jax.lax.dot_general(
        a, b, (((a.ndim - 1,), (0,)), ((), ())),
        preferred_element_type=jnp.float32,
        precision=jax.lax.Precision.HIGHEST)


def _encoder_body(x, w1, bb1, w2, bb2, out):
    t = bb1[...] * jnp.ones((x.shape[0], 1), jnp.float32)
    for k in range(4):
        t = t + (_bf(x[:, k:k + 1]) * _bf(w1[k:k + 1, :])).astype(jnp.float32)
    t = _relu(t)
    out[...] = _dot(t, w2[...]) + bb2[...]


def _encoder(x, We1, be1, We2, be2):
    return pl.pallas_call(
        _encoder_body,
        grid=(N // BN,),
        in_specs=[
            pl.BlockSpec((BN, 4), lambda i: (i, 0)),
            pl.BlockSpec((4, H), lambda i: (0, 0)),
            pl.BlockSpec((1, H), lambda i: (0, 0)),
            pl.BlockSpec((H, H), lambda i: (0, 0)),
            pl.BlockSpec((1, H), lambda i: (0, 0)),
        ],
        out_specs=pl.BlockSpec((BN, H), lambda i: (i, 0)),
        out_shape=jax.ShapeDtypeStruct((N, H), jnp.float32),
    )(x, We1, be1[None], We2, be2[None])


def _node_prep_body(h, w1d, w1s, bb1, a, b):
    hv = h[...]
    a[...] = _dot(hv, w1d[...]) + bb1[...]
    b[...] = _dot(hv, w1s[...])


def _node_prep(h, W1d, W1s, b1):
    return pl.pallas_call(
        _node_prep_body,
        grid=(N // BN,),
        in_specs=[
            pl.BlockSpec((BN, H), lambda i: (i, 0)),
            pl.BlockSpec((H, H), lambda i: (0, 0)),
            pl.BlockSpec((H, H), lambda i: (0, 0)),
            pl.BlockSpec((1, H), lambda i: (0, 0)),
        ],
        out_specs=[
            pl.BlockSpec((BN, H), lambda i: (i, 0)),
            pl.BlockSpec((BN, H), lambda i: (i, 0)),
        ],
        out_shape=[
            jax.ShapeDtypeStruct((N, H), jnp.float32),
            jax.ShapeDtypeStruct((N, H), jnp.float32),
        ],
    )(h, W1d, W1s, b1[None])


def _edge_mm_body(ag, bg, ea, w1e, w2, bb2, w3, bb3, u):
    g = ag[...] + bg[...]
    for k in range(3):
        g = g + (_bf(ea[:, k:k + 1]) * _bf(w1e[k:k + 1, :])).astype(jnp.float32)
    e = _dot(_relu(g), w2[...]) + bb2[...]
    t = _dot(e, w3[...]) + bb3[...]
    u[...] = _bf(_relu(t)).astype(jnp.float32)


def _edge_mm(Ag, Bg, eap, W1e, W2, b2, W3, b3):
    return pl.pallas_call(
        _edge_mm_body,
        grid=(EPAD // BM,),
        in_specs=[
            pl.BlockSpec((BM, H), lambda i: (i, 0)),
            pl.BlockSpec((BM, H), lambda i: (i, 0)),
            pl.BlockSpec((BM, 3), lambda i: (i, 0)),
            pl.BlockSpec((3, H), lambda i: (0, 0)),
            pl.BlockSpec((H, H), lambda i: (0, 0)),
            pl.BlockSpec((1, H), lambda i: (0, 0)),
            pl.BlockSpec((H, H), lambda i: (0, 0)),
            pl.BlockSpec((1, H), lambda i: (0, 0)),
        ],
        out_specs=pl.BlockSpec((BM, H), lambda i: (i, 0)),
        out_shape=jax.ShapeDtypeStruct((EPAD, H), jnp.float32),
    )(Ag, Bg, eap, W1e, W2, b2[None], W3, b3[None])


def _node_update_body(h, s, cnt, wua, wub, w4, bb4, bbu1, wu2, bbu2, g, b, out):
    hv = h[...]
    ssum = s[0] + s[1]
    c = cnt[0, :, 0:1] + cnt[1, :, 0:1]
    aggr = _dot_hi(ssum, _bf(w4[...]).astype(jnp.float32)) + c * bb4[...]
    z = (_dot(hv, wua[...])
         + _dot(aggr, wub[...])
         + bbu1[...])
    hn = _dot(_relu(z), wu2[...]) + bbu2[...]
    y = hv + hn
    m = jnp.mean(y, axis=-1, keepdims=True)
    yc = y - m
    v = jnp.mean(yc * yc, axis=-1, keepdims=True)
    out[...] = yc * lax.rsqrt(v + 1e-5) * g[...] + b[...]


def _node_update(h, S, cnt, Wua, Wub, W4, b4, bu1, Wu2, bu2, lg, lb):
    full = lambda i: (0, 0)
    return pl.pallas_call(
        _node_update_body,
        grid=(N // BN,),
        in_specs=[
            pl.BlockSpec((BN, H), lambda i: (i, 0)),
            pl.BlockSpec((2, BN, H), lambda i: (0, i, 0)),
            pl.BlockSpec((2, BN, H), lambda i: (0, i, 0)),
            pl.BlockSpec((H, H), full),
            pl.BlockSpec((H, H), full),
            pl.BlockSpec((H, H), full),
            pl.BlockSpec((1, H), full),
            pl.BlockSpec((1, H), full),
            pl.BlockSpec((H, H), full),
            pl.BlockSpec((1, H), full),
            pl.BlockSpec((1, H), full),
            pl.BlockSpec((1, H), full),
        ],
        out_specs=pl.BlockSpec((BN, H), lambda i: (i, 0)),
        out_shape=jax.ShapeDtypeStruct((N, H), jnp.float32),
    )(h, S, cnt, Wua, Wub, W4, b4[None], bu1[None], Wu2, bu2[None], lg[None],
      lb[None])


def _pool_body(h, ap0, ab0, ap1r, ga, gb1, gbm, gb2, gg, gbb, wk, bk, wv, bv,
               kout, vout):
    hv = h[...]
    t = jnp.tanh(_dot(hv, ap0[...]) + ab0[...])
    logits = jnp.sum((_bf(t) * _bf(ap1r[...])).astype(jnp.float32),
                     axis=-1, keepdims=True)
    m = jnp.max(logits)
    ew = jnp.exp(logits - m)
    s = jnp.sum(ew)
    w = ew / s
    hg = jnp.sum(hv * w, axis=0, keepdims=True)
    for i in range(3):
        u = _relu(_dot(hg, ga[i])
                  + gb1[i:i + 1, :])
        u = _dot(u, gbm[i]) + gb2[i:i + 1, :]
        y = hg + u
        mu = jnp.mean(y, axis=-1, keepdims=True)
        yc = y - mu
        var = jnp.mean(yc * yc, axis=-1, keepdims=True)
        hg = yc * lax.rsqrt(var + 1e-5) * gg[...] + gbb[...]
    kout[...] = _dot(hg, wk[...]) + bk[...]
    vout[...] = _dot(hg, wv[...]) + bv[...]


def _pool_global(h, attn_pool, gmlp, ln_global, kp, vp):
    ap0, ab0 = attn_pool[0]
    ap1 = attn_pool[1][0]
    ga = jnp.stack([mp[0][0] for mp in gmlp])
    gb1 = jnp.stack([mp[0][1] for mp in gmlp])
    gbm = jnp.stack([mp[1][0] for mp in gmlp])
    gb2 = jnp.stack([mp[1][1] for mp in gmlp])
    gg, gbb = ln_global
    full2 = lambda: None
    return pl.pallas_call(
        _pool_body,
        grid=(1,),
        in_specs=[
            pl.BlockSpec((N, H), lambda i: (0, 0)),
            pl.BlockSpec((H, H), lambda i: (0, 0)),
            pl.BlockSpec((1, H), lambda i: (0, 0)),
            pl.BlockSpec((1, H), lambda i: (0, 0)),
            pl.BlockSpec((3, H, H), lambda i: (0, 0, 0)),
            pl.BlockSpec((3, H), lambda i: (0, 0)),
            pl.BlockSpec((3, H, H), lambda i: (0, 0, 0)),
            pl.BlockSpec((3, H), lambda i: (0, 0)),
            pl.BlockSpec((1, H), lambda i: (0, 0)),
            pl.BlockSpec((1, H), lambda i: (0, 0)),
            pl.BlockSpec((H, H), lambda i: (0, 0)),
            pl.BlockSpec((1, H), lambda i: (0, 0)),
            pl.BlockSpec((H, H), lambda i: (0, 0)),
            pl.BlockSpec((1, H), lambda i: (0, 0)),
        ],
        out_specs=[
            pl.BlockSpec((1, H), lambda i: (0, 0)),
            pl.BlockSpec((1, H), lambda i: (0, 0)),
        ],
        out_shape=[
            jax.ShapeDtypeStruct((1, H), jnp.float32),
            jax.ShapeDtypeStruct((1, H), jnp.float32),
        ],
    )(h, ap0, ab0[None], ap1.T, ga, gb1, gbm, gb2, gg[None], gbb[None],
      kp[0], kp[1][None], vp[0], vp[1][None])


def _fuse_body(h, kv, vv, wq, bq, wo1, bo1, wo2, bo2, wh1, bh1, wh2, bh2,
               o0, o1, o2):
    hv = h[...]
    q = _dot(hv, wq[...]) + bq[...]
    s = jnp.sum(q * kv[...], axis=-1, keepdims=True) * (1.0 / np.sqrt(H))
    w = jax.nn.sigmoid(s)
    xo = hv + w * vv[...]
    y = _relu(_dot(xo, wo1[...]) + bo1[...])
    y = _dot(y, wo2[...]) + bo2[...]
    outs = [o0, o1, o2]
    for n in range(3):
        yh = _relu(_dot(y, wh1[n])
                   + bh1[n:n + 1, :])
        outs[n][...] = (jnp.sum((_bf(yh) * _bf(wh2[n:n + 1, :])).astype(
            jnp.float32), axis=-1, keepdims=True) + bh2[n:n + 1, :])


def _fuse_heads(h, Kv, Vv, qp, outp, heads):
    wh1 = jnp.stack([heads[n][0][0] for n in ('switch', 'voltage', 'current')])
    bh1 = jnp.stack([heads[n][0][1] for n in ('switch', 'voltage', 'current')])
    wh2 = jnp.stack([heads[n][1][0][:, 0] for n in ('switch', 'voltage', 'current')])
    bh2 = jnp.stack([heads[n][1][1] for n in ('switch', 'voltage', 'current')])
    full = lambda i: (0, 0)
    return pl.pallas_call(
        _fuse_body,
        grid=(N // BN,),
        in_specs=[
            pl.BlockSpec((BN, H), lambda i: (i, 0)),
            pl.BlockSpec((1, H), full),
            pl.BlockSpec((1, H), full),
            pl.BlockSpec((H, H), full),
            pl.BlockSpec((1, H), full),
            pl.BlockSpec((H, H), full),
            pl.BlockSpec((1, H), full),
            pl.BlockSpec((H, H), full),
            pl.BlockSpec((1, H), full),
            pl.BlockSpec((3, H, 64), lambda i: (0, 0, 0)),
            pl.BlockSpec((3, 64), full),
            pl.BlockSpec((3, 64), full),
            pl.BlockSpec((3, 1), full),
        ],
        out_specs=[
            pl.BlockSpec((BN, 1), lambda i: (i, 0)),
            pl.BlockSpec((BN, 1), lambda i: (i, 0)),
            pl.BlockSpec((BN, 1), lambda i: (i, 0)),
        ],
        out_shape=[
            jax.ShapeDtypeStruct((N, 1), jnp.float32),
            jax.ShapeDtypeStruct((N, 1), jnp.float32),
            jax.ShapeDtypeStruct((N, 1), jnp.float32),
        ],
    )(h, Kv, Vv, qp[0], qp[1][None], outp[0][0], outp[0][1][None],
      outp[1][0], outp[1][1][None], wh1, bh1, wh2, bh2)


def _sc_mesh():
    return plsc.VectorSubcoreMesh(core_axis_name="c", subcore_axis_name="s")


NCHP = 80


def _sc_gather_body(a_hbm, b_hbm, di_hbm, si_hbm, ag_hbm, bg_hbm,
                    idx_d, idx_s, ba0, ba1, bb0, bb1, sg0, sg1, sw0, sw1):
    wid = lax.axis_index("s") * 2 + lax.axis_index("c")
    base = wid * EPT
    pltpu.sync_copy(di_hbm.at[pl.ds(base, EPT)], idx_d)
    pltpu.sync_copy(si_hbm.at[pl.ds(base, EPT)], idx_s)
    bufa = (ba0, ba1)
    bufb = (bb0, bb1)
    sg = (sg0, sg1)
    sw = (sw0, sw1)

    def fire_g(j, k):
        jj = jnp.minimum(j, NCH - 1)
        sl = pl.ds(jj * CHUNK, CHUNK)
        pltpu.async_copy(a_hbm.at[idx_d.at[sl]], bufa[k], sg[k])
        pltpu.async_copy(b_hbm.at[idx_s.at[sl]], bufb[k], sg[k])

    def wait_g(k):
        pltpu.make_async_copy(a_hbm.at[pl.ds(0, CHUNK)], bufa[k], sg[k]).wait()
        pltpu.make_async_copy(a_hbm.at[pl.ds(0, CHUNK)], bufb[k], sg[k]).wait()

    def fire_w(j, k):
        jj = jnp.minimum(j, NCH - 1)
        sl = pl.ds(base + jj * CHUNK, CHUNK)
        pltpu.async_copy(bufa[k], ag_hbm.at[sl], sw[k])
        pltpu.async_copy(bufb[k], bg_hbm.at[sl], sw[k])

    def wait_w(k):
        pltpu.make_async_copy(a_hbm.at[pl.ds(0, CHUNK)], bufa[k], sw[k]).wait()
        pltpu.make_async_copy(a_hbm.at[pl.ds(0, CHUNK)], bufb[k], sw[k]).wait()

    fire_g(0, 0)

    def body(g, carry):
        for pz in (0, 1):
            j = 2 * g + pz

            @pl.when(j > 0)
            def _():
                wait_w(1 - pz)

            @pl.when(j + 1 < NCHP)
            def _():
                fire_g(j + 1, 1 - pz)

            wait_g(pz)
            fire_w(j, pz)
        return carry

    lax.fori_loop(0, NCHP // 2, body, 0)
    wait_w(1)


def _sc_gather(A, B, dstp, srcp):
    return pl.kernel(
        _sc_gather_body,
        out_type=(jax.ShapeDtypeStruct((EPAD, H), jnp.float32),
                  jax.ShapeDtypeStruct((EPAD, H), jnp.float32)),
        mesh=_sc_mesh(),
        scratch_types=[
            pltpu.VMEM((EPT,), jnp.int32),
            pltpu.VMEM((EPT,), jnp.int32),
            pltpu.VMEM((CHUNK, H), jnp.float32),
            pltpu.VMEM((CHUNK, H), jnp.float32),
            pltpu.VMEM((CHUNK, H), jnp.float32),
            pltpu.VMEM((CHUNK, H), jnp.float32),
            pltpu.SemaphoreType.DMA,
            pltpu.SemaphoreType.DMA,
            pltpu.SemaphoreType.DMA,
            pltpu.SemaphoreType.DMA,
        ],
    )(A, B, dstp, srcp)


def _sc_scatter_body(u_hbm, ds_hbm, z_hbm, s_hbm, idx2d, bu0, bu1, sl0, sl1,
                     acc_ref):
    cid = lax.axis_index("c")
    sid = lax.axis_index("s")
    wid = sid * 2 + cid
    rbase = sid * RPT
    pltpu.sync_copy(z_hbm.at[pl.ds(rbase, RPT)], acc_ref.at[pl.ds(rbase, RPT)])
    pltpu.sync_copy(ds_hbm.at[wid], idx2d)
    plsc.subcore_barrier()
    base = wid * EPT
    bufu = (bu0, bu1)
    sl = (sl0, sl1)

    def fire_l(j, k):
        jj = jnp.minimum(j, NCH - 1)
        pltpu.async_copy(u_hbm.at[pl.ds(base + jj * CHUNK, CHUNK)], bufu[k], sl[k])

    def wait_l(k):
        pltpu.make_async_copy(u_hbm.at[pl.ds(0, CHUNK)], bufu[k], sl[k]).wait()

    fire_l(0, 0)

    def body(g, carry):
        for pz in (0, 1):
            j = 2 * g + pz

            @pl.when(j + 1 < NCHP)
            def _():
                fire_l(j + 1, 1 - pz)

            wait_l(pz)
            pltpu.sync_copy(bufu[pz], acc_ref.at[idx2d.at[j]], add=True)
        return carry

    lax.fori_loop(0, NCHP // 2, body, 0)
    plsc.subcore_barrier()
    pltpu.sync_copy(acc_ref.at[pl.ds(rbase, RPT)],
                    s_hbm.at[cid].at[pl.ds(rbase, RPT)])


def _sc_scatter(U, dsts2, zf):
    return pl.kernel(
        _sc_scatter_body,
        out_type=jax.ShapeDtypeStruct((2, NP, H), jnp.float32),
        mesh=_sc_mesh(),
        scratch_types=[
            pltpu.VMEM((NCHP, CHUNK), jnp.int32),
            pltpu.VMEM((CHUNK, H), jnp.float32),
            pltpu.VMEM((CHUNK, H), jnp.float32),
            pltpu.SemaphoreType.DMA,
            pltpu.SemaphoreType.DMA,
            pltpu.VMEM_SHARED((NP, H), jnp.float32),
        ],
    )(U, dsts2, zf)


def _sc_cnt_body(ds_hbm, z_hbm, ones_hbm, cnt_hbm, idx_v, ones_v, acc):
    cid = lax.axis_index("c")
    sid = lax.axis_index("s")
    wid = sid * 2 + cid
    rbase = sid * RPT
    pltpu.sync_copy(z_hbm.at[pl.ds(rbase, RPT)], acc.at[pl.ds(rbase, RPT)])
    pltpu.sync_copy(ones_hbm, ones_v)
    plsc.subcore_barrier()
    base = wid * EPT

    def body(j, carry):
        off = base + j * CHUNK
        pltpu.sync_copy(ds_hbm.at[pl.ds(off, CHUNK)], idx_v)
        pltpu.sync_copy(ones_v, acc.at[idx_v], add=True)
        return carry

    lax.fori_loop(0, NCH, body, 0)
    plsc.subcore_barrier()
    pltpu.sync_copy(acc.at[pl.ds(rbase, RPT)], cnt_hbm.at[cid].at[pl.ds(rbase, RPT)])


def _sc_cnt(dsts, zc, ones):
    return pl.kernel(
        _sc_cnt_body,
        out_type=jax.ShapeDtypeStruct((2, NP, H), jnp.float32),
        mesh=_sc_mesh(),
        scratch_types=[
            pltpu.VMEM((CHUNK,), jnp.int32),
            pltpu.VMEM((CHUNK, H), jnp.float32),
            pltpu.VMEM_SHARED((NP, H), jnp.float32),
        ],
    )(dsts, zc, ones)


def kernel(x, edge_index, edge_attr, params):
    p = params
    src = edge_index[0]
    dst = edge_index[1]
    padz = jnp.zeros((EPAD - E,), jnp.int32)
    srcp = jnp.concatenate([src, padz])
    dstp = jnp.concatenate([dst, padz])
    dsts = jnp.concatenate([dst, jnp.full((EPAD - E,), N, jnp.int32)])
    dsts2 = jnp.concatenate(
        [dsts.reshape(NTILES, NCH, CHUNK),
         jnp.full((NTILES, 1, CHUNK), N, jnp.int32)], axis=1)
    eap = jnp.concatenate(
        [edge_attr, jnp.zeros((EPAD - E, 3), edge_attr.dtype)], axis=0)
    zf = jnp.zeros((NP, H), jnp.float32)
    ones128 = jnp.ones((CHUNK, H), jnp.float32)

    convs = p['convs']
    W1 = jnp.stack([c['edge'][0][0] for c in convs])
    b1 = jnp.stack([c['edge'][0][1] for c in convs])
    W2 = jnp.stack([c['edge'][1][0] for c in convs])
    b2 = jnp.stack([c['edge'][1][1] for c in convs])
    W3 = jnp.stack([c['msg'][0][0] for c in convs])
    b3 = jnp.stack([c['msg'][0][1] for c in convs])
    W4 = jnp.stack([c['msg'][1][0] for c in convs])
    b4 = jnp.stack([c['msg'][1][1] for c in convs])
    Wu = jnp.stack([c['upd'][0][0] for c in convs])
    bu1 = jnp.stack([c['upd'][0][1] for c in convs])
    Wu2 = jnp.stack([c['upd'][1][0] for c in convs])
    bu2 = jnp.stack([c['upd'][1][1] for c in convs])
    W1d = W1[:, :H]
    W1s = W1[:, H:2 * H]
    W1e = W1[:, 2 * H:]
    Wua = Wu[:, :H]
    Wub = Wu[:, H:]

    h = _encoder(x, p['enc'][0][0], p['enc'][0][1], p['enc'][1][0], p['enc'][1][1])

    cnt = _sc_cnt(dsts, zf, ones128)

    lg, lb = p['ln_local']
    for l in range(4):
        A, B = _node_prep(h, W1d[l], W1s[l], b1[l])
        Ag, Bg = _sc_gather(A, B, dstp, srcp)
        U = _edge_mm(Ag, Bg, eap, W1e[l], W2[l], b2[l], W3[l], b3[l])
        S = _sc_scatter(U, dsts2, zf)
        h = _node_update(h, S, cnt, Wua[l], Wub[l], W4[l], b4[l], bu1[l],
                         Wu2[l], bu2[l], lg, lb)

    Kv, Vv = _pool_global(h, p['attn_pool'], p['gmlp'], p['ln_global'],
                          p['k'], p['v'])
    o0, o1, o2 = _fuse_heads(h, Kv, Vv, p['q'], p['outp'], p['heads'])
    return jnp.concatenate([o0, o1, o2], axis=-1)
